# baseline (speedup 1.0000x reference)
import functools

import jax
import jax.numpy as jnp
from jax import lax
from jax.experimental import pallas as pl
from jax.experimental.pallas import tpu as pltpu
from jax.experimental.pallas import tpu_sc as plsc

VOCAB = 32000
EMB = 128
HID = 256
B = 32
S = 64
T = 64
G4 = 4 * HID
N_TOK = (S + T) * B
VT = 1280
TC = 16
NR = T // TC
NV = VOCAB // VT


def _gather_rows_sc(table, idx):
    info = plsc.get_sparse_core_info()
    nc, ns = info.num_cores, info.num_subcores
    nw = nc * ns
    per_w = N_TOK // nw
    mesh = plsc.VectorSubcoreMesh(core_axis_name="c", subcore_axis_name="s")

    @functools.partial(
        pl.kernel,
        mesh=mesh,
        out_type=jax.ShapeDtypeStruct((N_TOK, EMB), jnp.float32),
        scratch_types=[
            pltpu.VMEM((per_w,), jnp.int32),
            pltpu.VMEM((per_w, EMB), jnp.float32),
            pltpu.SemaphoreType.DMA,
        ],
    )
    def gk(table_hbm, idx_hbm, out_hbm, idx_v, rows_v, sem):
        wid = lax.axis_index("s") * nc + lax.axis_index("c")
        base = wid * per_w
        pltpu.sync_copy(idx_hbm.at[pl.ds(base, per_w)], idx_v)
        pltpu.async_copy(table_hbm.at[idx_v], rows_v, sem).wait()
        pltpu.sync_copy(rows_v, out_hbm.at[pl.ds(base, per_w)])

    return gk(table, idx)


def _nt_dot(a, b):
    return lax.dot_general(a, b, (((1,), (1,)), ((), ())),
                           preferred_element_type=jnp.float32)


def _cell(gates, c):
    sig = jax.nn.sigmoid(gates[:, : 3 * HID])
    i = sig[:, :HID]
    f = sig[:, HID: 2 * HID]
    o = sig[:, 2 * HID:]
    g = jnp.tanh(gates[:, 3 * HID:])
    c = f * c + i * g
    h = o * jnp.tanh(c)
    return h, c


def _fused_kernel(x_ref, ewih_ref, ewhh_ref, eb_ref, dwih_ref, dwhh_ref,
                  db_ref, fcw_ref, fcb_ref, o_ref,
                  xw_ref, xs_ref, h_ref, c_ref):
    v = pl.program_id(0)
    r = pl.program_id(1)

    @pl.when(jnp.logical_and(v == 0, r == 0))
    def _prologue():
        xb = x_ref[...].astype(jnp.bfloat16)
        xw_ref[: S * B] = _nt_dot(xb[: S * B], ewih_ref[...]) + eb_ref[...]
        xw_ref[S * B:] = _nt_dot(xb[S * B:], dwih_ref[...]) + db_ref[...]

        def enc_step(t, carry):
            h, c = carry
            gates = (xw_ref[pl.ds(t * B, B)]
                     + _nt_dot(h.astype(jnp.bfloat16), ewhh_ref[...]))
            return _cell(gates, c)

        zeros = jnp.zeros((B, HID), jnp.float32)
        h, c = lax.fori_loop(0, S, enc_step, (zeros, zeros))
        h_ref[...] = h
        c_ref[...] = c

    @pl.when(v == 0)
    def _decode_chunk():
        h = h_ref[...]
        c = c_ref[...]
        for j in range(TC):
            gates = (xw_ref[pl.ds((S + r * TC) * B + j * B, B)]
                     + _nt_dot(h.astype(jnp.bfloat16), dwhh_ref[...]))
            h, c = _cell(gates, c)
            xs_ref[:, r, j, :] = h
        h_ref[...] = h
        c_ref[...] = c

    xin = xs_ref[:, r, :, :].reshape(B * TC, HID)
    y = _nt_dot(xin.astype(jnp.bfloat16), fcw_ref[...]) + fcb_ref[...]
    o_ref[...] = y.reshape(B, TC, VT)


def _fused_call(x, ewih, ewhh, eb, dwih, dwhh, db, fc_w, fc_b2):
    full = lambda v, r: (0, 0)
    return pl.pallas_call(
        _fused_kernel,
        grid=(NV, NR),
        in_specs=[
            pl.BlockSpec((N_TOK, EMB), full),
            pl.BlockSpec((G4, EMB), full),
            pl.BlockSpec((G4, HID), full),
            pl.BlockSpec((1, G4), full),
            pl.BlockSpec((G4, EMB), full),
            pl.BlockSpec((G4, HID), full),
            pl.BlockSpec((1, G4), full),
            pl.BlockSpec((VT, HID), lambda v, r: (v, 0)),
            pl.BlockSpec((1, VT), lambda v, r: (0, v)),
        ],
        out_specs=pl.BlockSpec((B, TC, VT), lambda v, r: (0, r, v)),
        out_shape=jax.ShapeDtypeStruct((B, T, VOCAB), jnp.float32),
        scratch_shapes=[
            pltpu.VMEM((N_TOK, G4), jnp.float32),
            pltpu.VMEM((B, NR, TC, HID), jnp.float32),
            pltpu.VMEM((B, HID), jnp.float32),
            pltpu.VMEM((B, HID), jnp.float32),
        ],
    )(x, ewih, ewhh, eb, dwih, dwhh, db, fc_w, fc_b2)


def _permute_gates(w):
    i, f, g, o = jnp.split(w, 4, axis=0)
    return jnp.concatenate([i, f, o, g], axis=0)


def kernel(src, tgt, emb, enc_W_ih, enc_W_hh, enc_b_ih, enc_b_hh,
           dec_W_ih, dec_W_hh, dec_b_ih, dec_b_hh, fc_W, fc_b):
    idx = jnp.concatenate([src.T.reshape(-1), tgt.T.reshape(-1)])
    idx = idx.astype(jnp.int32)
    x = _gather_rows_sc(emb, idx)

    ewih = _permute_gates(enc_W_ih).astype(jnp.bfloat16)
    ewhh = _permute_gates(enc_W_hh).astype(jnp.bfloat16)
    eb = _permute_gates(enc_b_ih + enc_b_hh).reshape(1, G4)
    dwih = _permute_gates(dec_W_ih).astype(jnp.bfloat16)
    dwhh = _permute_gates(dec_W_hh).astype(jnp.bfloat16)
    db = _permute_gates(dec_b_ih + dec_b_hh).reshape(1, G4)

    return _fused_call(x, ewih, ewhh, eb, dwih, dwhh, db, fc_W,
                       fc_b.reshape(1, VOCAB))
